# baseline (speedup 1.0000x reference)
import dataclasses
import functools

import jax
import jax.numpy as jnp
from jax import lax
from jax.experimental import pallas as pl
from jax.experimental.pallas import tpu as pltpu
from jax.experimental.pallas import tpu_sc as plsc

NC = 2
NS = 16
NW = NC * NS
LANES = 16
TB = 2048
HB = TB // 2


def _tc_pack(table_t, V, E):
    nblk = (V + TB - 1) // TB

    def body(in_ref, out_ref):
        x = in_ref[...]
        xt = jnp.transpose(x)
        out_ref[:, 0:E] = xt[0:HB]
        out_ref[:, E : 2 * E] = xt[HB:TB]

    return pl.pallas_call(
        body,
        grid=(nblk,),
        in_specs=[pl.BlockSpec((E, TB), lambda i: (0, i))],
        out_specs=pl.BlockSpec((HB, 2 * E), lambda i: (i, 0)),
        out_shape=jax.ShapeDtypeStruct((nblk * HB, 2 * E), jnp.float32),
    )(table_t)


@functools.partial(jax.jit, static_argnums=(3, 4, 5))
def _sc_embed_t(ids_t, table2, pos, B, L, E):
    BW = B // NW
    NB = BW // LANES
    scale = 8.0

    mesh = plsc.VectorSubcoreMesh(core_axis_name="c", subcore_axis_name="s")

    @functools.partial(
        pl.kernel,
        out_type=jax.ShapeDtypeStruct((L, E // 8, B // 128, 8, 128), jnp.float32),
        mesh=mesh,
        scratch_types=[
            pltpu.VMEM((L, BW), jnp.int32),
            pltpu.VMEM((L, E), jnp.float32),
            pltpu.VMEM((BW,), jnp.int32),
            pltpu.VMEM((BW,), jnp.int32),
            pltpu.VMEM((BW, 2 * E), jnp.float32),
            pltpu.VMEM((BW, 2 * E), jnp.float32),
            pltpu.VMEM((E // 8, 8, BW), jnp.float32),
            pltpu.VMEM((E // 8, 8, BW), jnp.float32),
            pltpu.SemaphoreType.DMA,
            pltpu.SemaphoreType.DMA,
            pltpu.SemaphoreType.DMA,
            pltpu.SemaphoreType.DMA,
        ],
        compiler_params=dataclasses.replace(
            pltpu.CompilerParams(use_tc_tiling_on_sc=False),
            **(
                {"needs_layout_passes": False}
                if "needs_layout_passes" in pltpu.CompilerParams.__dataclass_fields__
                else {}
            ),
        ),
    )
    def k(table_hbm, ids_hbm, pos_hbm, out_hbm, ids_all, pos_v,
          idx0, idx1, rows0, rows1, outb0, outb1, sg0, sg1, so0, so1):
        idx_v = (idx0, idx1)
        rows_v = (rows0, rows1)
        out_v = (outb0, outb1)
        sg = (sg0, sg1)
        so = (so0, so1)

        wid = lax.axis_index("s") * NC + lax.axis_index("c")
        bbase = wid * BW

        pltpu.sync_copy(pos_hbm, pos_v)
        pltpu.sync_copy(ids_hbm.at[:, pl.ds(bbase, BW)], ids_all)

        def idx_prep(b, l):
            for c in range(NB):
                sl16 = pl.ds(c * LANES, LANES)
                tvec = ids_all[l, sl16]
                idx_v[b][sl16] = ((tvec >> 11) << 10) + (tvec & (HB - 1))

        def gather_fire(b):
            pltpu.async_copy(table_hbm.at[idx_v[b]], rows_v[b], sg[b])

        def gather_wait(b):
            pltpu.make_async_copy(table_hbm.at[idx_v[b]], rows_v[b], sg[b]).wait()

        def out_fire(b, l):
            pltpu.async_copy(out_v[b], out_hbm.at[l, :, wid], so[b])

        def out_wait(b):
            pltpu.make_async_copy(out_v[b], out_hbm.at[0, :, wid], so[b]).wait()

        def compute(b, l):
            for blk in range(NB):
                sl16 = pl.ds(blk * LANES, LANES)
                tvec = ids_all[l, sl16]
                mvec = jnp.where(tvec == 0, 0.0, 1.0)
                m8 = mvec * scale
                hvec = ((tvec >> 10) & 1) << 6
                rowvec = lax.iota(jnp.int32, LANES) + (blk * LANES)

                @pl.loop(0, E // LANES)
                def _(ec):
                    pos16 = pos_v[l, pl.ds(ec * LANES, LANES)]
                    for ei in range(LANES):
                        e = ec * LANES + ei
                        g = plsc.load_gather(rows_v[b], [rowvec, hvec + e])
                        et = ec * 2 + ei // 8
                        out_v[b][et, ei % 8, sl16] = g * m8 + pos16[ei] * mvec

        idx_prep(0, 0)
        gather_fire(0)

        @pl.loop(0, L // 2)
        def _(ss):
            for b in range(2):
                l = ss * 2 + b

                @pl.when(l + 1 < L)
                def _():
                    idx_prep(1 - b, l + 1)
                    gather_fire(1 - b)

                gather_wait(b)

                @pl.when(l >= 2)
                def _():
                    out_wait(b)

                compute(b, l)
                out_fire(b, l)

        out_wait(0)
        out_wait(1)

    return k(table2, ids_t, pos)


def kernel(inputs, token_table, position_table):
    B, L = inputs.shape
    V, E = token_table.shape
    ids_t = inputs.transpose(1, 0).astype(jnp.int32)
    table2 = _tc_pack(token_table.transpose(1, 0), V, E)
    out5 = _sc_embed_t(ids_t, table2, position_table, B, L, E)
    return out5.transpose(2, 4, 0, 1, 3).reshape(B, L, E)


# speedup vs baseline: 1.0648x; 1.0648x over previous
import dataclasses
import functools

import jax
import jax.numpy as jnp
from jax import lax
from jax.experimental import pallas as pl
from jax.experimental.pallas import tpu as pltpu
from jax.experimental.pallas import tpu_sc as plsc

NC = 2
NS = 16
NW = NC * NS
LANES = 16
TB = 8192
HB = TB // 2
TBS = TB.bit_length() - 1


def _tc_pack(table_t, V, E):
    nblk = (V + TB - 1) // TB

    def body(in_ref, out_ref):
        x = in_ref[...]
        xt = jnp.transpose(x)
        out_ref[:, 0:E] = xt[0:HB]
        out_ref[:, E : 2 * E] = xt[HB:TB]

    return pl.pallas_call(
        body,
        grid=(nblk,),
        in_specs=[pl.BlockSpec((E, TB), lambda i: (0, i))],
        out_specs=pl.BlockSpec((HB, 2 * E), lambda i: (i, 0)),
        out_shape=jax.ShapeDtypeStruct((nblk * HB, 2 * E), jnp.float32),
    )(table_t)


@functools.partial(jax.jit, static_argnums=(3, 4, 5))
def _sc_embed_t(ids_t, table2, pos, B, L, E):
    BW = B // NW
    NB = BW // LANES
    scale = 8.0

    mesh = plsc.VectorSubcoreMesh(core_axis_name="c", subcore_axis_name="s")

    @functools.partial(
        pl.kernel,
        out_type=jax.ShapeDtypeStruct((L, E // 8, B // 128, 8, 128), jnp.float32),
        mesh=mesh,
        scratch_types=[
            pltpu.VMEM((L, BW), jnp.int32),
            pltpu.VMEM((L, E), jnp.float32),
            pltpu.VMEM((BW,), jnp.int32),
            pltpu.VMEM((BW,), jnp.int32),
            pltpu.VMEM((BW, 2 * E), jnp.float32),
            pltpu.VMEM((BW, 2 * E), jnp.float32),
            pltpu.VMEM((E // 8, 8, BW), jnp.float32),
            pltpu.VMEM((E // 8, 8, BW), jnp.float32),
            pltpu.SemaphoreType.DMA,
            pltpu.SemaphoreType.DMA,
            pltpu.SemaphoreType.DMA,
            pltpu.SemaphoreType.DMA,
        ],
        compiler_params=dataclasses.replace(
            pltpu.CompilerParams(use_tc_tiling_on_sc=False),
            **(
                {"needs_layout_passes": False}
                if "needs_layout_passes" in pltpu.CompilerParams.__dataclass_fields__
                else {}
            ),
        ),
    )
    def k(table_hbm, ids_hbm, pos_hbm, out_hbm, ids_all, pos_v,
          idx0, idx1, rows0, rows1, outb0, outb1, sg0, sg1, so0, so1):
        idx_v = (idx0, idx1)
        rows_v = (rows0, rows1)
        out_v = (outb0, outb1)
        sg = (sg0, sg1)
        so = (so0, so1)

        wid = lax.axis_index("s") * NC + lax.axis_index("c")
        bbase = wid * BW

        pltpu.sync_copy(pos_hbm, pos_v)
        pltpu.sync_copy(ids_hbm.at[:, pl.ds(bbase, BW)], ids_all)

        def idx_prep(b, l):
            for c in range(NB):
                sl16 = pl.ds(c * LANES, LANES)
                tvec = ids_all[l, sl16]
                idx_v[b][sl16] = ((tvec >> TBS) << (TBS - 1)) + (tvec & (HB - 1))

        def gather_fire(b):
            pltpu.async_copy(table_hbm.at[idx_v[b]], rows_v[b], sg[b])

        def gather_wait(b):
            pltpu.make_async_copy(table_hbm.at[idx_v[b]], rows_v[b], sg[b]).wait()

        def out_fire(b, l):
            pltpu.async_copy(out_v[b], out_hbm.at[l, :, wid], so[b])

        def out_wait(b):
            pltpu.make_async_copy(out_v[b], out_hbm.at[0, :, wid], so[b]).wait()

        iota16 = lax.iota(jnp.int32, LANES)
        one = jnp.ones((LANES,), jnp.int32)

        def compute(b, l):
            @pl.loop(0, E // LANES)
            def _(ec):
                pos16 = pos_v[l, pl.ds(ec * LANES, LANES)]
                psplat = [pos16[ei] + jnp.zeros((LANES,), jnp.float32)
                          for ei in range(LANES)]
                for blk in range(NB):
                    sl16 = pl.ds(blk * LANES, LANES)
                    tvec = ids_all[l, sl16]
                    mvec = jnp.where(tvec == 0, 0.0, 1.0)
                    hvec = ((tvec >> (TBS - 1)) & 1) << 6
                    rowvec = iota16 + (blk * LANES)
                    col = hvec + (ec * LANES)
                    for ei in range(LANES):
                        g = plsc.load_gather(rows_v[b], [rowvec, col])
                        et = ec * 2 + ei // 8
                        out_v[b][et, ei % 8, sl16] = (g * scale + psplat[ei]) * mvec
                        if ei + 1 < LANES:
                            col = col + one

        idx_prep(0, 0)
        gather_fire(0)

        @pl.loop(0, L // 2)
        def _(ss):
            for b in range(2):
                l = ss * 2 + b

                @pl.when(l + 1 < L)
                def _():
                    idx_prep(1 - b, l + 1)
                    gather_fire(1 - b)

                gather_wait(b)

                @pl.when(l >= 2)
                def _():
                    out_wait(b)

                compute(b, l)
                out_fire(b, l)

        out_wait(0)
        out_wait(1)

    return k(table2, ids_t, pos)


def kernel(inputs, token_table, position_table):
    B, L = inputs.shape
    V, E = token_table.shape
    ids_t = inputs.transpose(1, 0).astype(jnp.int32)
    table2 = _tc_pack(token_table.transpose(1, 0), V, E)
    out5 = _sc_embed_t(ids_t, table2, position_table, B, L, E)
    return out5.transpose(2, 4, 0, 1, 3).reshape(B, L, E)


# speedup vs baseline: 1.7095x; 1.6055x over previous
import dataclasses
import functools

import jax
import jax.numpy as jnp
from jax import lax
from jax.experimental import pallas as pl
from jax.experimental.pallas import tpu as pltpu
from jax.experimental.pallas import tpu_sc as plsc

NC = 2
NS = 16
NW = NC * NS
LANES = 16
TB = 8192
HB = TB // 2
TBS = TB.bit_length() - 1


def _tc_pack(table_t, V, E):
    nblk = (V + TB - 1) // TB

    def body(in_ref, out_ref):
        x = in_ref[...]
        xt = jnp.transpose(x)
        out_ref[:, 0:E] = xt[0:HB]
        out_ref[:, E : 2 * E] = xt[HB:TB]

    return pl.pallas_call(
        body,
        grid=(nblk,),
        in_specs=[pl.BlockSpec((E, TB), lambda i: (0, i))],
        out_specs=pl.BlockSpec((HB, 2 * E), lambda i: (i, 0)),
        out_shape=jax.ShapeDtypeStruct((nblk * HB, 2 * E), jnp.float32),
    )(table_t)


@functools.partial(jax.jit, static_argnums=(3, 4, 5))
def _sc_embed(ids, table2, position_table, B, L, E):
    steps = B // NW
    n_ids = steps * L
    scale = 8.0

    mesh = plsc.VectorSubcoreMesh(core_axis_name="c", subcore_axis_name="s")

    @functools.partial(
        pl.kernel,
        out_type=jax.ShapeDtypeStruct((B * L, 2 * E), jnp.float32),
        mesh=mesh,
        scratch_types=[
            pltpu.VMEM((n_ids,), jnp.int32),
            pltpu.VMEM((L, E), jnp.float32),
            pltpu.VMEM((L,), jnp.int32),
            pltpu.VMEM((L,), jnp.int32),
            pltpu.VMEM((L, 2 * E), jnp.float32),
            pltpu.VMEM((L, 2 * E), jnp.float32),
            pltpu.VMEM((L, E), jnp.float32),
            pltpu.VMEM((L, E), jnp.float32),
            pltpu.SemaphoreType.DMA,
            pltpu.SemaphoreType.DMA,
            pltpu.SemaphoreType.DMA,
            pltpu.SemaphoreType.DMA,
        ],
        compiler_params=dataclasses.replace(
            pltpu.CompilerParams(use_tc_tiling_on_sc=False),
            **(
                {"needs_layout_passes": False}
                if "needs_layout_passes" in pltpu.CompilerParams.__dataclass_fields__
                else {}
            ),
        ),
    )
    def k(table_hbm, ids_hbm, pos_hbm, out_hbm, ids_all, pos_v,
          idx0, idx1, rows0, rows1, outb0, outb1, sg0, sg1, so0, so1):
        idx_v = (idx0, idx1)
        rows_v = (rows0, rows1)
        out_v = (outb0, outb1)
        sg = (sg0, sg1)
        so = (so0, so1)

        wid = lax.axis_index("s") * NC + lax.axis_index("c")
        wbase = wid * n_ids

        pltpu.sync_copy(pos_hbm, pos_v)
        pltpu.sync_copy(ids_hbm.at[pl.ds(wbase, n_ids)], ids_all)

        def idx_prep(b, sl):
            for c in range(L // LANES):
                sl16 = pl.ds(c * LANES, LANES)
                tvec = ids_all[pl.ds(sl * L + c * LANES, LANES)]
                idx_v[b][sl16] = ((tvec >> TBS) << (TBS - 1)) + (tvec & (HB - 1))
            if L % LANES:
                o = L - LANES
                tvec = ids_all[pl.ds(sl * L + o, LANES)]
                idx_v[b][pl.ds(o, LANES)] = ((tvec >> TBS) << (TBS - 1)) + (
                    tvec & (HB - 1)
                )

        g_chunks = [(o, min(128, L - o)) for o in range(0, L, 128)]

        def gather_fire(b):
            for o, n in g_chunks:
                pltpu.async_copy(
                    table_hbm.at[idx_v[b].at[pl.ds(o, n)]],
                    rows_v[b].at[pl.ds(o, n)],
                    sg[b],
                )

        def gather_wait(b):
            for o, n in g_chunks:
                pltpu.make_async_copy(
                    table_hbm.at[idx_v[b].at[pl.ds(o, n)]],
                    rows_v[b].at[pl.ds(o, n)],
                    sg[b],
                ).wait()

        def out_fire(b, sl):
            pltpu.async_copy(
                out_v[b],
                out_hbm.at[pl.ds(wbase + sl * L, L), pl.ds(0, E)],
                so[b],
            )

        def out_wait(b):
            pltpu.make_async_copy(
                out_v[b],
                out_hbm.at[pl.ds(0, L), pl.ds(0, E)],
                so[b],
            ).wait()

        def compute(b, sl):
            def do_rows(b16, j_lo):
                idvec = ids_all[pl.ds(sl * L + b16, LANES)]
                mvec = jnp.where(idvec == 0, 0.0, 1.0)
                hvec = ((idvec >> (TBS - 1)) & 1) << 6
                for j in range(j_lo, LANES):
                    m = mvec[j]
                    h = hvec[j]
                    w = b16 + j
                    for c in range(E // LANES):
                        sl16 = pl.ds(c * LANES, LANES)
                        out_v[b][w, sl16] = (
                            rows_v[b][w, pl.ds(h + c * LANES, LANES)] * scale
                            + pos_v[w, sl16]
                        ) * m

            @pl.loop(0, L // LANES)
            def _(g):
                do_rows(g * LANES, 0)

            if L % LANES:
                do_rows(L - LANES, LANES - L % LANES)

        idx_prep(0, 0)
        gather_fire(0)

        @pl.loop(0, steps // 2)
        def _(ss):
            for b in range(2):
                sl = ss * 2 + b

                @pl.when(sl + 1 < steps)
                def _():
                    idx_prep(1 - b, sl + 1)
                    gather_fire(1 - b)

                gather_wait(b)

                @pl.when(sl >= 2)
                def _():
                    out_wait(b)

                compute(b, sl)
                out_fire(b, sl)

        out_wait(0)
        out_wait(1)

    return k(table2, ids, position_table)


def kernel(inputs, token_table, position_table):
    B, L = inputs.shape
    V, E = token_table.shape
    ids = inputs.reshape(-1).astype(jnp.int32)
    table2 = _tc_pack(token_table.transpose(1, 0), V, E)
    out2 = _sc_embed(ids, table2, position_table, B, L, E)
    return out2[:, :E].reshape(B, L, E)


# speedup vs baseline: 2.0908x; 1.2231x over previous
import functools

import jax
import jax.numpy as jnp
from jax import lax
from jax.experimental import pallas as pl
from jax.experimental.pallas import tpu as pltpu
from jax.experimental.pallas import tpu_sc as plsc

NC = 2
NS = 16
NW = NC * NS
LANES = 16


@functools.partial(jax.jit, static_argnums=(3, 4, 5))
def _sc_embed(ids, token_table, position_table, B, L, E):
    steps = B // NW
    n_ids = steps * L
    scale = 8.0

    mesh = plsc.VectorSubcoreMesh(core_axis_name="c", subcore_axis_name="s")

    @functools.partial(
        pl.kernel,
        out_type=jax.ShapeDtypeStruct((B * L, 2 * E), jnp.float32),
        mesh=mesh,
        scratch_types=[
            pltpu.VMEM((n_ids,), jnp.int32),
            pltpu.VMEM((L, E), jnp.float32),
            pltpu.VMEM((L, E), jnp.float32),
            pltpu.VMEM((L, E), jnp.float32),
            pltpu.VMEM((L, E), jnp.float32),
            pltpu.VMEM((L, E), jnp.float32),
            pltpu.SemaphoreType.DMA,
            pltpu.SemaphoreType.DMA,
            pltpu.SemaphoreType.DMA,
            pltpu.SemaphoreType.DMA,
        ],
        compiler_params=pltpu.CompilerParams(use_tc_tiling_on_sc=False),
    )
    def k(table_hbm, ids_hbm, pos_hbm, out_hbm, ids_all, pos_v,
          rows0, rows1, outb0, outb1, sg0, sg1, so0, so1):
        rows_v = (rows0, rows1)
        out_v = (outb0, outb1)
        sg = (sg0, sg1)
        so = (so0, so1)

        wid = lax.axis_index("s") * NC + lax.axis_index("c")
        wbase = wid * n_ids

        pltpu.sync_copy(pos_hbm, pos_v)
        pltpu.sync_copy(ids_hbm.at[pl.ds(wbase, n_ids)], ids_all)

        g_chunks = [(o, min(128, L - o)) for o in range(0, L, 128)]

        def gather_fire(b, sl):
            for o, n in g_chunks:
                pltpu.async_copy(
                    table_hbm.at[ids_all.at[pl.ds(sl * L + o, n)]],
                    rows_v[b].at[pl.ds(o, n)],
                    sg[b],
                )

        def gather_wait(b):
            for o, n in g_chunks:
                pltpu.make_async_copy(
                    table_hbm.at[ids_all.at[pl.ds(o, n)]],
                    rows_v[b].at[pl.ds(o, n)],
                    sg[b],
                ).wait()

        def out_fire(b, sl):
            pltpu.async_copy(
                out_v[b],
                out_hbm.at[pl.ds(wbase + sl * L, L), pl.ds(0, E)],
                so[b],
            )

        def out_wait(b):
            pltpu.make_async_copy(
                out_v[b],
                out_hbm.at[pl.ds(0, L), pl.ds(0, E)],
                so[b],
            ).wait()

        def compute(b, sl):
            def do_rows(b16, j_lo):
                idvec = ids_all[pl.ds(sl * L + b16, LANES)]
                mvec = jnp.where(idvec == 0, 0.0, 1.0)
                for j in range(j_lo, LANES):
                    m = mvec[j]
                    w = b16 + j
                    for c in range(E // LANES):
                        sl16 = pl.ds(c * LANES, LANES)
                        out_v[b][w, sl16] = (
                            rows_v[b][w, sl16] * scale + pos_v[w, sl16]
                        ) * m

            @pl.loop(0, L // LANES)
            def _(g):
                do_rows(g * LANES, 0)

            if L % LANES:
                do_rows(L - LANES, LANES - L % LANES)

        gather_fire(0, 0)

        @pl.loop(0, steps // 2)
        def _(ss):
            for b in range(2):
                sl = ss * 2 + b

                @pl.when(sl + 1 < steps)
                def _():
                    gather_fire(1 - b, sl + 1)

                gather_wait(b)

                @pl.when(sl >= 2)
                def _():
                    out_wait(b)

                compute(b, sl)
                out_fire(b, sl)

        out_wait(0)
        out_wait(1)

    return k(token_table, ids, position_table)


def kernel(inputs, token_table, position_table):
    B, L = inputs.shape
    V, E = token_table.shape
    ids = inputs.reshape(-1).astype(jnp.int32)
    out2 = _sc_embed(ids, token_table, position_table, B, L, E)
    return out2[:, :E].reshape(B, L, E)
